# Initial kernel scaffold; baseline (speedup 1.0000x reference)
import jax
import jax.numpy as jnp
from jax.experimental import pallas as pl


def kernel(node_feat, edge_feat, edge_index, W0, b0, W1, b1, W2, b2, W3, b3, W4, b4, Wr1, br1, Wr2, br2, Wr3, br3):
    raise NotImplementedError("write your pallas kernel here")


# speedup vs baseline: 8.0235x; 8.0235x over previous
import functools

import jax
import jax.numpy as jnp
from jax import lax
from jax.experimental import pallas as pl
from jax.experimental.pallas import tpu as pltpu
from jax.experimental.pallas import tpu_sc as plsc

_CHUNK = 2048
_NW = 32
_SLICE = 6272
_NPAD = 16 * _SLICE


def _sc_segment_sum(dst_r, val_r):
    n_chunks = dst_r.shape[0]
    iters = (n_chunks + _NW - 1) // _NW
    mesh = plsc.VectorSubcoreMesh(core_axis_name="c", subcore_axis_name="s")

    @functools.partial(
        pl.kernel,
        mesh=mesh,
        out_type=(
            jax.ShapeDtypeStruct((_NPAD,), jnp.float32),
            jax.ShapeDtypeStruct((_NPAD,), jnp.float32),
        ),
        scratch_types=[
            pltpu.VMEM((_CHUNK,), jnp.int32),
            pltpu.VMEM((_CHUNK,), jnp.float32),
            pltpu.VMEM((_SLICE,), jnp.float32),
            pltpu.VMEM_SHARED((_NPAD,), jnp.float32),
        ],
    )
    def seg_sum(dst_hbm, val_hbm, out0_hbm, out1_hbm, idx_v, val_v, stage_v, acc_sh):
        cid = lax.axis_index("c")
        sid = lax.axis_index("s")
        w = sid * 2 + cid

        def zero_body(i, carry):
            stage_v[pl.ds(i * 16, 16)] = jnp.zeros((16,), jnp.float32)
            return carry

        lax.fori_loop(0, _SLICE // 16, zero_body, 0)
        pltpu.sync_copy(stage_v, acc_sh.at[pl.ds(sid * _SLICE, _SLICE)])
        plsc.subcore_barrier()

        def body(i, carry):
            chunk = w + i * _NW

            @pl.when(chunk < n_chunks)
            def _():
                pltpu.sync_copy(dst_hbm.at[chunk], idx_v)
                pltpu.sync_copy(val_hbm.at[chunk], val_v)
                pltpu.sync_copy(val_v, acc_sh.at[idx_v], add=True)

            return carry

        lax.fori_loop(0, iters, body, 0)
        plsc.subcore_barrier()

        pltpu.sync_copy(acc_sh.at[pl.ds(sid * _SLICE, _SLICE)], stage_v)

        @pl.when(cid == 0)
        def _():
            pltpu.sync_copy(stage_v, out0_hbm.at[pl.ds(sid * _SLICE, _SLICE)])

        @pl.when(cid == 1)
        def _():
            pltpu.sync_copy(stage_v, out1_hbm.at[pl.ds(sid * _SLICE, _SLICE)])

    return seg_sum(dst_r, val_r)


def _tc_mlp_body(x_ref, he0_ref, he1_ref,
                 m0, m1, m2, m3, m4, mr1, mr2, mr3,
                 l0, l1, l2, l3, l4, lr1,
                 c0, c1, c2, c3, c4, cr1, cr2, cr3,
                 o_ref):
    he = he0_ref[...] + he1_ref[...]
    x = x_ref[...]
    for m, l, c in ((m0, l0, c0), (m1, l1, c1), (m2, l2, c2),
                    (m3, l3, c3), (m4, l4, c4)):
        acc = jnp.dot(x, m[...], preferred_element_type=jnp.float32)
        x = jnp.maximum(acc + he * l[...] + c[...], 0.0)
    acc = jnp.dot(x, mr1[...], preferred_element_type=jnp.float32)
    x = jnp.maximum(acc + he * lr1[...] + cr1[...], 0.0)
    x = jnp.maximum(
        jnp.dot(x, mr2[...], preferred_element_type=jnp.float32) + cr2[...], 0.0)
    o_ref[...] = (
        jnp.dot(x, mr3[...], preferred_element_type=jnp.float32) + cr3[...])


def kernel(node_feat, edge_feat, edge_index, W0, b0, W1, b1, W2, b2, W3, b3,
           W4, b4, Wr1, br1, Wr2, br2, Wr3, br3):
    N, D = node_feat.shape
    E = edge_feat.shape[0]
    REG = Wr3.shape[0]

    n_chunks = E // _CHUNK
    dst_r = edge_index[1].reshape(n_chunks, _CHUNK)
    val_r = edge_feat.reshape(E).reshape(n_chunks, _CHUNK)

    hp0, hp1 = _sc_segment_sum(dst_r, val_r)
    he0 = hp0.reshape(_NPAD, 1)
    he1 = hp1.reshape(_NPAD, 1)

    sage = [(W0, b0), (W1, b1), (W2, b2), (W3, b3), (W4, b4), (Wr1, br1)]
    mats = [jnp.transpose(W[:, :D]) for W, _ in sage]
    lasts = [W[:, D].reshape(1, D) for W, _ in sage]
    biases = [b.reshape(1, D) for _, b in sage]
    m2 = jnp.transpose(Wr2)
    m3 = jnp.transpose(Wr3)
    c2 = br2.reshape(1, D)
    c3 = br3.reshape(1, REG)

    BN = 4000
    grid = (N // BN,)
    full = lambda i: (0, 0)
    row = lambda i: (i, 0)
    in_specs = (
        [pl.BlockSpec((BN, D), row),
         pl.BlockSpec((BN, 1), row),
         pl.BlockSpec((BN, 1), row)]
        + [pl.BlockSpec((D, D), full)] * 7
        + [pl.BlockSpec((D, REG), full)]
        + [pl.BlockSpec((1, D), full)] * 13
        + [pl.BlockSpec((1, REG), full)]
    )
    out = pl.pallas_call(
        _tc_mlp_body,
        grid=grid,
        in_specs=in_specs,
        out_specs=pl.BlockSpec((BN, REG), row),
        out_shape=jax.ShapeDtypeStruct((N, REG), jnp.float32),
        compiler_params=pltpu.CompilerParams(
            dimension_semantics=("parallel",)),
    )(node_feat, he0, he1,
      mats[0], mats[1], mats[2], mats[3], mats[4], mats[5], m2, m3,
      lasts[0], lasts[1], lasts[2], lasts[3], lasts[4], lasts[5],
      biases[0], biases[1], biases[2], biases[3], biases[4], biases[5],
      c2, c3)
    return out


# speedup vs baseline: 20.6457x; 2.5732x over previous
import functools

import jax
import jax.numpy as jnp
from jax import lax
from jax.experimental import pallas as pl
from jax.experimental.pallas import tpu as pltpu
from jax.experimental.pallas import tpu_sc as plsc

_CHUNK = 2048
_NW = 32
_SLICE = 6272
_NPAD = 16 * _SLICE


def _sc_segment_sum(edge_index, ef_flat):
    n_chunks = ef_flat.shape[0] // _CHUNK
    iters = (n_chunks + _NW - 1) // _NW
    mesh = plsc.VectorSubcoreMesh(core_axis_name="c", subcore_axis_name="s")

    @functools.partial(
        pl.kernel,
        mesh=mesh,
        out_type=(
            jax.ShapeDtypeStruct((_NPAD,), jnp.float32),
            jax.ShapeDtypeStruct((_NPAD,), jnp.float32),
        ),
        scratch_types=[
            pltpu.VMEM((_CHUNK,), jnp.int32),
            pltpu.VMEM((_CHUNK,), jnp.float32),
            pltpu.VMEM((_SLICE,), jnp.float32),
            pltpu.VMEM_SHARED((_NPAD,), jnp.float32),
        ],
    )
    def seg_sum(dst_hbm, val_hbm, out0_hbm, out1_hbm, idx_v, val_v, stage_v, acc_sh):
        cid = lax.axis_index("c")
        sid = lax.axis_index("s")
        w = sid * 2 + cid

        def zero_body(i, carry):
            stage_v[pl.ds(i * 16, 16)] = jnp.zeros((16,), jnp.float32)
            return carry

        lax.fori_loop(0, _SLICE // 16, zero_body, 0)
        pltpu.sync_copy(stage_v, acc_sh.at[pl.ds(sid * _SLICE, _SLICE)])
        plsc.subcore_barrier()

        def body(i, carry):
            chunk = w + i * _NW

            @pl.when(chunk < n_chunks)
            def _():
                pltpu.sync_copy(dst_hbm.at[1, pl.ds(chunk * _CHUNK, _CHUNK)], idx_v)
                pltpu.sync_copy(val_hbm.at[pl.ds(chunk * _CHUNK, _CHUNK)], val_v)
                pltpu.sync_copy(val_v, acc_sh.at[idx_v], add=True)

            return carry

        lax.fori_loop(0, iters, body, 0)
        plsc.subcore_barrier()

        pltpu.sync_copy(acc_sh.at[pl.ds(sid * _SLICE, _SLICE)], stage_v)

        @pl.when(cid == 0)
        def _():
            pltpu.sync_copy(stage_v, out0_hbm.at[pl.ds(sid * _SLICE, _SLICE)])

        @pl.when(cid == 1)
        def _():
            pltpu.sync_copy(stage_v, out1_hbm.at[pl.ds(sid * _SLICE, _SLICE)])

    return seg_sum(edge_index, ef_flat)


def _tc_mlp_body(x_ref, he0_ref, he1_ref,
                 m0, m1, m2, m3, m4, mr1, mr2, mr3,
                 l0, l1, l2, l3, l4, lr1,
                 c0, c1, c2, c3, c4, cr1, cr2, cr3,
                 o_ref):
    he = he0_ref[...] + he1_ref[...]
    x = x_ref[...]
    for m, l, c in ((m0, l0, c0), (m1, l1, c1), (m2, l2, c2),
                    (m3, l3, c3), (m4, l4, c4)):
        acc = jnp.dot(x, m[...], preferred_element_type=jnp.float32)
        x = jnp.maximum(acc + he * l[...] + c[...], 0.0)
    acc = jnp.dot(x, mr1[...], preferred_element_type=jnp.float32)
    x = jnp.maximum(acc + he * lr1[...] + cr1[...], 0.0)
    x = jnp.maximum(
        jnp.dot(x, mr2[...], preferred_element_type=jnp.float32) + cr2[...], 0.0)
    o_ref[...] = (
        jnp.dot(x, mr3[...], preferred_element_type=jnp.float32) + cr3[...])


def kernel(node_feat, edge_feat, edge_index, W0, b0, W1, b1, W2, b2, W3, b3,
           W4, b4, Wr1, br1, Wr2, br2, Wr3, br3):
    N, D = node_feat.shape
    E = edge_feat.shape[0]
    REG = Wr3.shape[0]

    hp0, hp1 = _sc_segment_sum(edge_index, edge_feat.reshape(E))
    he0 = hp0.reshape(_NPAD, 1)
    he1 = hp1.reshape(_NPAD, 1)

    sage = [(W0, b0), (W1, b1), (W2, b2), (W3, b3), (W4, b4), (Wr1, br1)]
    mats = [jnp.transpose(W[:, :D]) for W, _ in sage]
    lasts = [W[:, D].reshape(1, D) for W, _ in sage]
    biases = [b.reshape(1, D) for _, b in sage]
    m2 = jnp.transpose(Wr2)
    m3 = jnp.transpose(Wr3)
    c2 = br2.reshape(1, D)
    c3 = br3.reshape(1, REG)

    BN = 4000
    grid = (N // BN,)
    full = lambda i: (0, 0)
    row = lambda i: (i, 0)
    in_specs = (
        [pl.BlockSpec((BN, D), row),
         pl.BlockSpec((BN, 1), row),
         pl.BlockSpec((BN, 1), row)]
        + [pl.BlockSpec((D, D), full)] * 7
        + [pl.BlockSpec((D, REG), full)]
        + [pl.BlockSpec((1, D), full)] * 13
        + [pl.BlockSpec((1, REG), full)]
    )
    out = pl.pallas_call(
        _tc_mlp_body,
        grid=grid,
        in_specs=in_specs,
        out_specs=pl.BlockSpec((BN, REG), row),
        out_shape=jax.ShapeDtypeStruct((N, REG), jnp.float32),
        compiler_params=pltpu.CompilerParams(
            dimension_semantics=("parallel",)),
    )(node_feat, he0, he1,
      mats[0], mats[1], mats[2], mats[3], mats[4], mats[5], m2, m3,
      lasts[0], lasts[1], lasts[2], lasts[3], lasts[4], lasts[5],
      biases[0], biases[1], biases[2], biases[3], biases[4], biases[5],
      c2, c3)
    return out


# speedup vs baseline: 25.4224x; 1.2314x over previous
import functools

import jax
import jax.numpy as jnp
from jax import lax
from jax.experimental import pallas as pl
from jax.experimental.pallas import tpu as pltpu
from jax.experimental.pallas import tpu_sc as plsc

_CHUNK = 2048
_NW = 32
_SLICE = 6272
_NPAD = 16 * _SLICE


def _sc_segment_sum(edge_index, ef_flat):
    n_chunks = ef_flat.shape[0] // _CHUNK
    iters = (n_chunks + _NW - 1) // _NW
    mesh = plsc.VectorSubcoreMesh(core_axis_name="c", subcore_axis_name="s")

    @functools.partial(
        pl.kernel,
        mesh=mesh,
        out_type=(
            jax.ShapeDtypeStruct((_NPAD,), jnp.float32),
            jax.ShapeDtypeStruct((_NPAD,), jnp.float32),
        ),
        scratch_types=[
            pltpu.VMEM((2, _CHUNK), jnp.int32),
            pltpu.VMEM((_CHUNK,), jnp.float32),
            pltpu.VMEM((2, _CHUNK), jnp.int32),
            pltpu.VMEM((_CHUNK,), jnp.float32),
            pltpu.VMEM((_CHUNK,), jnp.int32),
            pltpu.VMEM((_CHUNK,), jnp.int32),
            pltpu.VMEM((_SLICE,), jnp.float32),
            pltpu.VMEM_SHARED((_NPAD,), jnp.float32),
            pltpu.SemaphoreType.DMA,
            pltpu.SemaphoreType.DMA,
        ],
    )
    def seg_sum(dst_hbm, val_hbm, out0_hbm, out1_hbm,
                idx0, val0, idx1, val1, idxf0, idxf1, stage_v, acc_sh,
                sem0, sem1):
        cid = lax.axis_index("c")
        sid = lax.axis_index("s")
        w = sid * 2 + cid

        def start_loads(chunk, idx_v, val_v, sem):
            pltpu.async_copy(
                dst_hbm.at[:, pl.ds(chunk * _CHUNK, _CHUNK)], idx_v, sem)
            pltpu.async_copy(
                val_hbm.at[pl.ds(chunk * _CHUNK, _CHUNK)], val_v, sem)

        def wait_loads(idx_v, val_v, sem):
            pltpu.make_async_copy(
                dst_hbm.at[:, pl.ds(0, _CHUNK)], idx_v, sem).wait()
            pltpu.make_async_copy(
                val_hbm.at[pl.ds(0, _CHUNK)], val_v, sem).wait()

        def zero_body(i, carry):
            stage_v[pl.ds(i * 16, 16)] = jnp.zeros((16,), jnp.float32)
            return carry

        lax.fori_loop(0, _SLICE // 16, zero_body, 0)
        pltpu.sync_copy(stage_v, acc_sh.at[pl.ds(sid * _SLICE, _SLICE)])
        plsc.subcore_barrier()

        start_loads(w, idx0, val0, sem0)

        def pair(k2, carry):
            g0 = w + (k2 * 2) * _NW
            g1 = g0 + _NW
            g2 = g1 + _NW

            @pl.when(g1 < n_chunks)
            def _():
                start_loads(g1, idx1, val1, sem1)

            @pl.when(g0 < n_chunks)
            def _():
                wait_loads(idx0, val0, sem0)

                def cp0(i, carry):
                    idxf0[pl.ds(i * 16, 16)] = idx0[1, pl.ds(i * 16, 16)]
                    return carry

                lax.fori_loop(0, _CHUNK // 16, cp0, 0)
                pltpu.sync_copy(val0, acc_sh.at[idxf0], add=True)

            @pl.when(g2 < n_chunks)
            def _():
                start_loads(g2, idx0, val0, sem0)

            @pl.when(g1 < n_chunks)
            def _():
                wait_loads(idx1, val1, sem1)

                def cp1(i, carry):
                    idxf1[pl.ds(i * 16, 16)] = idx1[1, pl.ds(i * 16, 16)]
                    return carry

                lax.fori_loop(0, _CHUNK // 16, cp1, 0)
                pltpu.sync_copy(val1, acc_sh.at[idxf1], add=True)

            return carry

        lax.fori_loop(0, (iters + 1) // 2, pair, 0)
        plsc.subcore_barrier()

        pltpu.sync_copy(acc_sh.at[pl.ds(sid * _SLICE, _SLICE)], stage_v)

        @pl.when(cid == 0)
        def _():
            pltpu.sync_copy(stage_v, out0_hbm.at[pl.ds(sid * _SLICE, _SLICE)])

        @pl.when(cid == 1)
        def _():
            pltpu.sync_copy(stage_v, out1_hbm.at[pl.ds(sid * _SLICE, _SLICE)])

    return seg_sum(edge_index, ef_flat)


def _tc_mlp_body(x_ref, he0_ref, he1_ref,
                 m0, m1, m2, m3, m4, mr1, mr2, mr3,
                 l0, l1, l2, l3, l4, lr1,
                 c0, c1, c2, c3, c4, cr1, cr2, cr3,
                 o_ref):
    he = he0_ref[...] + he1_ref[...]
    x = x_ref[...]
    for m, l, c in ((m0, l0, c0), (m1, l1, c1), (m2, l2, c2),
                    (m3, l3, c3), (m4, l4, c4)):
        acc = jnp.dot(x, m[...], preferred_element_type=jnp.float32)
        x = jnp.maximum(acc + he * l[...] + c[...], 0.0)
    acc = jnp.dot(x, mr1[...], preferred_element_type=jnp.float32)
    x = jnp.maximum(acc + he * lr1[...] + cr1[...], 0.0)
    x = jnp.maximum(
        jnp.dot(x, mr2[...], preferred_element_type=jnp.float32) + cr2[...], 0.0)
    o_ref[...] = (
        jnp.dot(x, mr3[...], preferred_element_type=jnp.float32) + cr3[...])


def kernel(node_feat, edge_feat, edge_index, W0, b0, W1, b1, W2, b2, W3, b3,
           W4, b4, Wr1, br1, Wr2, br2, Wr3, br3):
    N, D = node_feat.shape
    E = edge_feat.shape[0]
    REG = Wr3.shape[0]

    hp0, hp1 = _sc_segment_sum(edge_index, edge_feat.reshape(E))
    he0 = hp0.reshape(_NPAD, 1)
    he1 = hp1.reshape(_NPAD, 1)

    sage = [(W0, b0), (W1, b1), (W2, b2), (W3, b3), (W4, b4), (Wr1, br1)]
    mats = [jnp.transpose(W[:, :D]) for W, _ in sage]
    lasts = [W[:, D].reshape(1, D) for W, _ in sage]
    biases = [b.reshape(1, D) for _, b in sage]
    m2 = jnp.transpose(Wr2)
    m3 = jnp.transpose(Wr3)
    c2 = br2.reshape(1, D)
    c3 = br3.reshape(1, REG)

    BN = 4000
    grid = (N // BN,)
    full = lambda i: (0, 0)
    row = lambda i: (i, 0)
    in_specs = (
        [pl.BlockSpec((BN, D), row),
         pl.BlockSpec((BN, 1), row),
         pl.BlockSpec((BN, 1), row)]
        + [pl.BlockSpec((D, D), full)] * 7
        + [pl.BlockSpec((D, REG), full)]
        + [pl.BlockSpec((1, D), full)] * 13
        + [pl.BlockSpec((1, REG), full)]
    )
    out = pl.pallas_call(
        _tc_mlp_body,
        grid=grid,
        in_specs=in_specs,
        out_specs=pl.BlockSpec((BN, REG), row),
        out_shape=jax.ShapeDtypeStruct((N, REG), jnp.float32),
        compiler_params=pltpu.CompilerParams(
            dimension_semantics=("parallel",)),
    )(node_feat, he0, he1,
      mats[0], mats[1], mats[2], mats[3], mats[4], mats[5], m2, m3,
      lasts[0], lasts[1], lasts[2], lasts[3], lasts[4], lasts[5],
      biases[0], biases[1], biases[2], biases[3], biases[4], biases[5],
      c2, c3)
    return out


# speedup vs baseline: 26.8745x; 1.0571x over previous
import functools

import jax
import jax.numpy as jnp
from jax import lax
from jax.experimental import pallas as pl
from jax.experimental.pallas import tpu as pltpu
from jax.experimental.pallas import tpu_sc as plsc

_CHUNK = 2048
_NW = 32
_SLICE = 6272
_NPAD = 16 * _SLICE

_DIMS = (((1,), (1,)), ((), ()))


def _sc_segment_sum(edge_index, ef_flat):
    n_chunks = ef_flat.shape[0] // _CHUNK
    iters = (n_chunks + _NW - 1) // _NW
    mesh = plsc.VectorSubcoreMesh(core_axis_name="c", subcore_axis_name="s")

    @functools.partial(
        pl.kernel,
        mesh=mesh,
        out_type=(
            jax.ShapeDtypeStruct((_NPAD,), jnp.float32),
            jax.ShapeDtypeStruct((_NPAD,), jnp.float32),
        ),
        scratch_types=[
            pltpu.VMEM((2, _CHUNK), jnp.int32),
            pltpu.VMEM((_CHUNK,), jnp.float32),
            pltpu.VMEM((2, _CHUNK), jnp.int32),
            pltpu.VMEM((_CHUNK,), jnp.float32),
            pltpu.VMEM((_CHUNK,), jnp.int32),
            pltpu.VMEM((_CHUNK,), jnp.int32),
            pltpu.VMEM((_SLICE,), jnp.float32),
            pltpu.VMEM_SHARED((_NPAD,), jnp.float32),
            pltpu.SemaphoreType.DMA,
            pltpu.SemaphoreType.DMA,
        ],
    )
    def seg_sum(dst_hbm, val_hbm, out0_hbm, out1_hbm,
                idx0, val0, idx1, val1, idxf0, idxf1, stage_v, acc_sh,
                sem0, sem1):
        cid = lax.axis_index("c")
        sid = lax.axis_index("s")
        w = sid * 2 + cid
        my_rows = pl.ds(sid * _SLICE, _SLICE)

        def start_loads(chunk, idx_v, val_v, sem):
            pltpu.async_copy(
                dst_hbm.at[:, pl.ds(chunk * _CHUNK, _CHUNK)], idx_v, sem)
            pltpu.async_copy(
                val_hbm.at[pl.ds(chunk * _CHUNK, _CHUNK)], val_v, sem)

        def wait_loads(idx_v, val_v, sem):
            pltpu.make_async_copy(
                dst_hbm.at[:, pl.ds(0, _CHUNK)], idx_v, sem).wait()
            pltpu.make_async_copy(
                val_hbm.at[pl.ds(0, _CHUNK)], val_v, sem).wait()

        def zero_body(i, carry):
            stage_v[pl.ds(i * 16, 16)] = jnp.zeros((16,), jnp.float32)
            return carry

        lax.fori_loop(0, _SLICE // 16, zero_body, 0)
        pltpu.sync_copy(stage_v, acc_sh.at[my_rows])
        plsc.subcore_barrier()

        start_loads(w, idx0, val0, sem0)

        def pair(k2, carry):
            g0 = w + (k2 * 2) * _NW
            g1 = g0 + _NW
            g2 = g1 + _NW

            @pl.when(g1 < n_chunks)
            def _():
                start_loads(g1, idx1, val1, sem1)

            @pl.when(g0 < n_chunks)
            def _():
                wait_loads(idx0, val0, sem0)

                def cp0(i, c):
                    idxf0[pl.ds(i * 16, 16)] = idx0[1, pl.ds(i * 16, 16)]
                    return c

                lax.fori_loop(0, _CHUNK // 16, cp0, 0)
                pltpu.sync_copy(val0, acc_sh.at[idxf0], add=True)

            @pl.when(g2 < n_chunks)
            def _():
                start_loads(g2, idx0, val0, sem0)

            @pl.when(g1 < n_chunks)
            def _():
                wait_loads(idx1, val1, sem1)

                def cp1(i, c):
                    idxf1[pl.ds(i * 16, 16)] = idx1[1, pl.ds(i * 16, 16)]
                    return c

                lax.fori_loop(0, _CHUNK // 16, cp1, 0)
                pltpu.sync_copy(val1, acc_sh.at[idxf1], add=True)

            return carry

        lax.fori_loop(0, (iters + 1) // 2, pair, 0)
        plsc.subcore_barrier()

        pltpu.sync_copy(acc_sh.at[my_rows], stage_v)

        @pl.when(cid == 0)
        def _():
            pltpu.sync_copy(stage_v, out0_hbm.at[my_rows])

        @pl.when(cid == 1)
        def _():
            pltpu.sync_copy(stage_v, out1_hbm.at[my_rows])

    return seg_sum(edge_index, ef_flat)


def _tc_mlp_body(x_ref, he0_ref, he1_ref,
                 w0, w1, w2, w3, w4, wr1, wr2, wr3,
                 c0, c1, c2, c3, c4, cr1, cr2, cr3,
                 o_ref):
    d = x_ref.shape[1]
    bn = x_ref.shape[0]
    he = (he0_ref[...] + he1_ref[...]).reshape(1, bn)
    x = x_ref[...]
    outer_dims = (((0,), (1,)), ((), ()))
    for wref, cref in ((w0, c0), (w1, c1), (w2, c2), (w3, c3), (w4, c4),
                       (wr1, cr1)):
        wfull = wref[...]
        y = lax.dot_general(x, wfull[:, :d], _DIMS,
                            preferred_element_type=jnp.float32)
        y = y + lax.dot_general(he, wfull[:, d:d + 1], outer_dims,
                                preferred_element_type=jnp.float32)
        x = jnp.maximum(y + cref[...].reshape(1, d), 0.0)
    x = jnp.maximum(
        lax.dot_general(x, wr2[...], _DIMS, preferred_element_type=jnp.float32)
        + cr2[...].reshape(1, d), 0.0)
    o_ref[...] = (
        lax.dot_general(x, wr3[...], _DIMS, preferred_element_type=jnp.float32)
        + cr3[...].reshape(1, wr3.shape[0]))


def kernel(node_feat, edge_feat, edge_index, W0, b0, W1, b1, W2, b2, W3, b3,
           W4, b4, Wr1, br1, Wr2, br2, Wr3, br3):
    N, D = node_feat.shape
    REG = Wr3.shape[0]

    he0, he1 = _sc_segment_sum(edge_index, edge_feat.reshape(-1))

    BN = 4096
    grid = (pl.cdiv(N, BN),)
    full = lambda i: (0, 0)
    vec = lambda i: (0,)
    row = lambda i: (i, 0)
    blk = lambda i: (i,)
    in_specs = (
        [pl.BlockSpec((BN, D), row),
         pl.BlockSpec((BN,), blk),
         pl.BlockSpec((BN,), blk)]
        + [pl.BlockSpec((D, D + 1), full)] * 6
        + [pl.BlockSpec((D, D), full)]
        + [pl.BlockSpec((REG, D), full)]
        + [pl.BlockSpec((D,), vec)] * 7
        + [pl.BlockSpec((REG,), vec)]
    )
    out = pl.pallas_call(
        _tc_mlp_body,
        grid=grid,
        in_specs=in_specs,
        out_specs=pl.BlockSpec((BN, REG), row),
        out_shape=jax.ShapeDtypeStruct((N, REG), jnp.float32),
        compiler_params=pltpu.CompilerParams(
            dimension_semantics=("parallel",)),
    )(node_feat, he0, he1,
      W0, W1, W2, W3, W4, Wr1, Wr2, Wr3,
      b0, b1, b2, b3, b4, br1, br2, br3)
    return out


# speedup vs baseline: 33.8491x; 1.2595x over previous
import functools

import jax
import jax.numpy as jnp
from jax import lax
from jax.experimental import pallas as pl
from jax.experimental.pallas import tpu as pltpu
from jax.experimental.pallas import tpu_sc as plsc

_CHUNK = 6400
_NW = 32
_SLICE = 6272
_NPAD = 16 * _SLICE

_DIMS = (((1,), (1,)), ((), ()))
_OUTER = (((0,), (0,)), ((), ()))


def _sc_segment_sum(edge_index, ef_flat):
    n_chunks = ef_flat.shape[0] // _CHUNK
    iters = (n_chunks + _NW - 1) // _NW
    mesh = plsc.VectorSubcoreMesh(core_axis_name="c", subcore_axis_name="s")

    @functools.partial(
        pl.kernel,
        mesh=mesh,
        out_type=(
            jax.ShapeDtypeStruct((_NPAD,), jnp.float32),
            jax.ShapeDtypeStruct((_NPAD,), jnp.float32),
        ),
        scratch_types=[
            pltpu.VMEM((2, _CHUNK), jnp.int32),
            pltpu.VMEM((_CHUNK,), jnp.float32),
            pltpu.VMEM((2, _CHUNK), jnp.int32),
            pltpu.VMEM((_CHUNK,), jnp.float32),
            pltpu.VMEM((_CHUNK,), jnp.int32),
            pltpu.VMEM((_CHUNK,), jnp.int32),
            pltpu.VMEM((_SLICE,), jnp.float32),
            pltpu.VMEM_SHARED((_NPAD,), jnp.float32),
            pltpu.SemaphoreType.DMA,
            pltpu.SemaphoreType.DMA,
        ],
    )
    def seg_sum(dst_hbm, val_hbm, out0_hbm, out1_hbm,
                idx0, val0, idx1, val1, idxf0, idxf1, stage_v, acc_sh,
                sem0, sem1):
        cid = lax.axis_index("c")
        sid = lax.axis_index("s")
        w = sid * 2 + cid
        my_rows = pl.ds(sid * _SLICE, _SLICE)

        def start_loads(chunk, idx_v, val_v, sem):
            pltpu.async_copy(
                dst_hbm.at[:, pl.ds(chunk * _CHUNK, _CHUNK)], idx_v, sem)
            pltpu.async_copy(
                val_hbm.at[pl.ds(chunk * _CHUNK, _CHUNK)], val_v, sem)

        def wait_loads(idx_v, val_v, sem):
            pltpu.make_async_copy(
                dst_hbm.at[:, pl.ds(0, _CHUNK)], idx_v, sem).wait()
            pltpu.make_async_copy(
                val_hbm.at[pl.ds(0, _CHUNK)], val_v, sem).wait()

        def extract_row(idx_v, idxf_v):
            def cp(i, c):
                for j in range(8):
                    o = (i * 8 + j) * 16
                    idxf_v[pl.ds(o, 16)] = idx_v[1, pl.ds(o, 16)]
                return c

            lax.fori_loop(0, _CHUNK // 128, cp, 0)

        def zero_body(i, carry):
            stage_v[pl.ds(i * 16, 16)] = jnp.zeros((16,), jnp.float32)
            return carry

        lax.fori_loop(0, _SLICE // 16, zero_body, 0)
        pltpu.sync_copy(stage_v, acc_sh.at[my_rows])
        plsc.subcore_barrier()

        start_loads(w, idx0, val0, sem0)

        def pair(k2, carry):
            g0 = w + (k2 * 2) * _NW
            g1 = g0 + _NW
            g2 = g1 + _NW

            @pl.when(g1 < n_chunks)
            def _():
                start_loads(g1, idx1, val1, sem1)

            @pl.when(g0 < n_chunks)
            def _():
                wait_loads(idx0, val0, sem0)
                extract_row(idx0, idxf0)
                pltpu.sync_copy(val0, acc_sh.at[idxf0], add=True)

            @pl.when(g2 < n_chunks)
            def _():
                start_loads(g2, idx0, val0, sem0)

            @pl.when(g1 < n_chunks)
            def _():
                wait_loads(idx1, val1, sem1)
                extract_row(idx1, idxf1)
                pltpu.sync_copy(val1, acc_sh.at[idxf1], add=True)

            return carry

        lax.fori_loop(0, (iters + 1) // 2, pair, 0)
        plsc.subcore_barrier()

        pltpu.sync_copy(acc_sh.at[my_rows], stage_v)

        @pl.when(cid == 0)
        def _():
            pltpu.sync_copy(stage_v, out0_hbm.at[my_rows])

        @pl.when(cid == 1)
        def _():
            pltpu.sync_copy(stage_v, out1_hbm.at[my_rows])

    return seg_sum(edge_index, ef_flat)


_P_WR2 = 6 * 128
_P_WL = _P_WR2 + 128
_P_B = _P_WL + 6


def _tc_mlp_body(x_ref, he0_ref, he1_ref, p_ref, wr3_ref, o_ref):
    d = x_ref.shape[1]
    bn = x_ref.shape[0]
    he = (he0_ref[...] + he1_ref[...]).reshape(bn, 1)
    x = x_ref[...]
    for l in range(6):
        y = jnp.dot(x, p_ref[l * d:(l + 1) * d, :],
                    preferred_element_type=jnp.float32)
        y = y + he * p_ref[_P_WL + l:_P_WL + l + 1, :]
        x = jnp.maximum(y + p_ref[_P_B + l:_P_B + l + 1, :], 0.0)
    x = jnp.maximum(
        jnp.dot(x, p_ref[_P_WR2:_P_WR2 + d, :],
                preferred_element_type=jnp.float32)
        + p_ref[_P_B + 6:_P_B + 7, :], 0.0)
    reg = wr3_ref.shape[0]
    o_ref[...] = (
        lax.dot_general(x, wr3_ref[...], _DIMS,
                        preferred_element_type=jnp.float32)
        + p_ref[_P_B + 7:_P_B + 8, :reg])


def kernel(node_feat, edge_feat, edge_index, W0, b0, W1, b1, W2, b2, W3, b3,
           W4, b4, Wr1, br1, Wr2, br2, Wr3, br3):
    N, D = node_feat.shape
    REG = Wr3.shape[0]

    he0, he1 = _sc_segment_sum(edge_index, edge_feat.reshape(-1))

    sage_w = (W0, W1, W2, W3, W4, Wr1)
    pack = ([jnp.transpose(W[:, :D]) for W in sage_w]
            + [jnp.transpose(Wr2)]
            + [W[:, D].reshape(1, D) for W in sage_w]
            + [b.reshape(1, D) for b in (b0, b1, b2, b3, b4, br1, br2)]
            + [jnp.pad(br3, (0, D - REG)).reshape(1, D)])
    P = jnp.concatenate(pack, axis=0)

    BN = 4096
    grid = (pl.cdiv(N, BN),)
    in_specs = [
        pl.BlockSpec((BN, D), lambda i: (i, 0)),
        pl.BlockSpec((BN,), lambda i: (i,)),
        pl.BlockSpec((BN,), lambda i: (i,)),
        pl.BlockSpec(P.shape, lambda i: (0, 0)),
        pl.BlockSpec((REG, D), lambda i: (0, 0)),
    ]
    out = pl.pallas_call(
        _tc_mlp_body,
        grid=grid,
        in_specs=in_specs,
        out_specs=pl.BlockSpec((BN, REG), lambda i: (i, 0)),
        out_shape=jax.ShapeDtypeStruct((N, REG), jnp.float32),
        compiler_params=pltpu.CompilerParams(
            dimension_semantics=("parallel",)),
    )(node_feat, he0, he1, P, Wr3)
    return out


# speedup vs baseline: 33.8877x; 1.0011x over previous
import functools

import jax
import jax.numpy as jnp
from jax import lax
from jax.experimental import pallas as pl
from jax.experimental.pallas import tpu as pltpu
from jax.experimental.pallas import tpu_sc as plsc

_CHUNK = 6400
_NW = 32
_SLICE = 6272
_NPAD = 16 * _SLICE

_DIMS = (((1,), (1,)), ((), ()))
_OUTER = (((0,), (0,)), ((), ()))


def _sc_segment_sum(edge_index, ef_flat):
    n_chunks = ef_flat.shape[0] // _CHUNK
    iters = (n_chunks + _NW - 1) // _NW
    mesh = plsc.VectorSubcoreMesh(core_axis_name="c", subcore_axis_name="s")

    @functools.partial(
        pl.kernel,
        mesh=mesh,
        out_type=(
            jax.ShapeDtypeStruct((_NPAD,), jnp.float32),
            jax.ShapeDtypeStruct((_NPAD,), jnp.float32),
        ),
        scratch_types=[
            pltpu.VMEM((2, _CHUNK), jnp.int32),
            pltpu.VMEM((_CHUNK,), jnp.float32),
            pltpu.VMEM((2, _CHUNK), jnp.int32),
            pltpu.VMEM((_CHUNK,), jnp.float32),
            pltpu.VMEM((_CHUNK,), jnp.int32),
            pltpu.VMEM((_CHUNK,), jnp.int32),
            pltpu.VMEM((_SLICE,), jnp.float32),
            pltpu.VMEM_SHARED((_NPAD,), jnp.float32),
            pltpu.SemaphoreType.DMA,
            pltpu.SemaphoreType.DMA,
        ],
    )
    def seg_sum(dst_hbm, val_hbm, out0_hbm, out1_hbm,
                idx0, val0, idx1, val1, idxf0, idxf1, stage_v, acc_sh,
                sem0, sem1):
        cid = lax.axis_index("c")
        sid = lax.axis_index("s")
        w = sid * 2 + cid
        my_rows = pl.ds(sid * _SLICE, _SLICE)

        def start_loads(chunk, idx_v, val_v, sem):
            pltpu.async_copy(
                dst_hbm.at[:, pl.ds(chunk * _CHUNK, _CHUNK)], idx_v, sem)
            pltpu.async_copy(
                val_hbm.at[pl.ds(chunk * _CHUNK, _CHUNK)], val_v, sem)

        def wait_loads(idx_v, val_v, sem):
            pltpu.make_async_copy(
                dst_hbm.at[:, pl.ds(0, _CHUNK)], idx_v, sem).wait()
            pltpu.make_async_copy(
                val_hbm.at[pl.ds(0, _CHUNK)], val_v, sem).wait()

        def extract_row(idx_v, idxf_v):
            def cp(i, c):
                for j in range(8):
                    o = (i * 8 + j) * 16
                    idxf_v[pl.ds(o, 16)] = idx_v[1, pl.ds(o, 16)]
                return c

            lax.fori_loop(0, _CHUNK // 128, cp, 0)

        def zero_body(i, carry):
            stage_v[pl.ds(i * 16, 16)] = jnp.zeros((16,), jnp.float32)
            return carry

        lax.fori_loop(0, _SLICE // 16, zero_body, 0)
        pltpu.sync_copy(stage_v, acc_sh.at[my_rows])
        plsc.subcore_barrier()

        start_loads(w, idx0, val0, sem0)

        def pair(k2, carry):
            g0 = w + (k2 * 2) * _NW
            g1 = g0 + _NW
            g2 = g1 + _NW

            @pl.when(g1 < n_chunks)
            def _():
                start_loads(g1, idx1, val1, sem1)

            @pl.when(g0 < n_chunks)
            def _():
                wait_loads(idx0, val0, sem0)
                extract_row(idx0, idxf0)
                pltpu.sync_copy(val0, acc_sh.at[idxf0], add=True)

            @pl.when(g2 < n_chunks)
            def _():
                start_loads(g2, idx0, val0, sem0)

            @pl.when(g1 < n_chunks)
            def _():
                wait_loads(idx1, val1, sem1)
                extract_row(idx1, idxf1)
                pltpu.sync_copy(val1, acc_sh.at[idxf1], add=True)

            return carry

        lax.fori_loop(0, (iters + 1) // 2, pair, 0)
        plsc.subcore_barrier()

        pltpu.sync_copy(acc_sh.at[my_rows], stage_v)

        @pl.when(cid == 0)
        def _():
            pltpu.sync_copy(stage_v, out0_hbm.at[my_rows])

        @pl.when(cid == 1)
        def _():
            pltpu.sync_copy(stage_v, out1_hbm.at[my_rows])

    return seg_sum(edge_index, ef_flat)


_P_WR2 = 6 * 128
_P_WL = _P_WR2 + 128
_P_B = _P_WL + 6


def _tc_mlp_body(x_ref, he0_ref, he1_ref, p_ref, wr3_ref, o_ref):
    d = x_ref.shape[1]
    bn = x_ref.shape[0]
    he = (he0_ref[...] + he1_ref[...]).reshape(bn, 1)
    x = x_ref[...]
    for l in range(6):
        y = jnp.dot(x, p_ref[l * d:(l + 1) * d, :],
                    preferred_element_type=jnp.float32)
        y = y + he * p_ref[_P_WL + l:_P_WL + l + 1, :]
        x = jnp.maximum(y + p_ref[_P_B + l:_P_B + l + 1, :], 0.0)
    x = jnp.maximum(
        jnp.dot(x, p_ref[_P_WR2:_P_WR2 + d, :],
                preferred_element_type=jnp.float32)
        + p_ref[_P_B + 6:_P_B + 7, :], 0.0)
    reg = wr3_ref.shape[0]
    o_ref[:, :reg] = (
        lax.dot_general(x, wr3_ref[...], _DIMS,
                        preferred_element_type=jnp.float32)
        + p_ref[_P_B + 7:_P_B + 8, :reg])


def kernel(node_feat, edge_feat, edge_index, W0, b0, W1, b1, W2, b2, W3, b3,
           W4, b4, Wr1, br1, Wr2, br2, Wr3, br3):
    N, D = node_feat.shape
    REG = Wr3.shape[0]

    he0, he1 = _sc_segment_sum(edge_index, edge_feat.reshape(-1))

    sage_w = (W0, W1, W2, W3, W4, Wr1)
    pack = ([jnp.transpose(W[:, :D]) for W in sage_w]
            + [jnp.transpose(Wr2)]
            + [W[:, D].reshape(1, D) for W in sage_w]
            + [b.reshape(1, D) for b in (b0, b1, b2, b3, b4, br1, br2)]
            + [jnp.pad(br3, (0, D - REG)).reshape(1, D)])
    P = jnp.concatenate(pack, axis=0)

    BN = 4096
    grid = (pl.cdiv(N, BN),)
    in_specs = [
        pl.BlockSpec((BN, D), lambda i: (i, 0)),
        pl.BlockSpec((BN,), lambda i: (i,)),
        pl.BlockSpec((BN,), lambda i: (i,)),
        pl.BlockSpec(P.shape, lambda i: (0, 0)),
        pl.BlockSpec((REG, D), lambda i: (0, 0)),
    ]
    out = pl.pallas_call(
        _tc_mlp_body,
        grid=grid,
        in_specs=in_specs,
        out_specs=pl.BlockSpec((BN, D), lambda i: (i, 0)),
        out_shape=jax.ShapeDtypeStruct((N, D), jnp.float32),
        compiler_params=pltpu.CompilerParams(
            dimension_semantics=("parallel",)),
    )(node_feat, he0, he1, P, Wr3)
    return out[:, :REG]


# speedup vs baseline: 38.6377x; 1.1402x over previous
import functools

import jax
import jax.numpy as jnp
from jax import lax
from jax.experimental import pallas as pl
from jax.experimental.pallas import tpu as pltpu
from jax.experimental.pallas import tpu_sc as plsc

_CHUNK = 6400
_NW = 32
_SLICE = 6272
_NPAD = 16 * _SLICE

_DIMS = (((1,), (1,)), ((), ()))
_OUTER = (((0,), (0,)), ((), ()))


def _sc_segment_sum(edge_index, ef_flat):
    n_chunks = ef_flat.shape[0] // _CHUNK
    iters = (n_chunks + _NW - 1) // _NW
    mesh = plsc.VectorSubcoreMesh(core_axis_name="c", subcore_axis_name="s")

    @functools.partial(
        pl.kernel,
        mesh=mesh,
        out_type=(
            jax.ShapeDtypeStruct((_NPAD,), jnp.float32),
            jax.ShapeDtypeStruct((_NPAD,), jnp.float32),
        ),
        scratch_types=[
            pltpu.VMEM((2, _CHUNK), jnp.int32),
            pltpu.VMEM((_CHUNK,), jnp.float32),
            pltpu.VMEM((2, _CHUNK), jnp.int32),
            pltpu.VMEM((_CHUNK,), jnp.float32),
            pltpu.VMEM((_CHUNK,), jnp.int32),
            pltpu.VMEM((_CHUNK,), jnp.int32),
            pltpu.VMEM((_SLICE,), jnp.float32),
            pltpu.VMEM_SHARED((_NPAD,), jnp.float32),
            pltpu.SemaphoreType.DMA,
            pltpu.SemaphoreType.DMA,
        ],
    )
    def seg_sum(dst_hbm, val_hbm, out0_hbm, out1_hbm,
                idx0, val0, idx1, val1, idxf0, idxf1, stage_v, acc_sh,
                sem0, sem1):
        cid = lax.axis_index("c")
        sid = lax.axis_index("s")
        w = sid * 2 + cid
        my_rows = pl.ds(sid * _SLICE, _SLICE)

        def start_loads(chunk, idx_v, val_v, sem):
            pltpu.async_copy(
                dst_hbm.at[:, pl.ds(chunk * _CHUNK, _CHUNK)], idx_v, sem)
            pltpu.async_copy(
                val_hbm.at[pl.ds(chunk * _CHUNK, _CHUNK)], val_v, sem)

        def wait_loads(idx_v, val_v, sem):
            pltpu.make_async_copy(
                dst_hbm.at[:, pl.ds(0, _CHUNK)], idx_v, sem).wait()
            pltpu.make_async_copy(
                val_hbm.at[pl.ds(0, _CHUNK)], val_v, sem).wait()

        def extract_row(idx_v, idxf_v):
            def cp(i, c):
                for j in range(8):
                    o = (i * 8 + j) * 16
                    idxf_v[pl.ds(o, 16)] = idx_v[1, pl.ds(o, 16)]
                return c

            lax.fori_loop(0, _CHUNK // 128, cp, 0)

        def zero_body(i, carry):
            stage_v[pl.ds(i * 16, 16)] = jnp.zeros((16,), jnp.float32)
            return carry

        lax.fori_loop(0, _SLICE // 16, zero_body, 0)
        pltpu.sync_copy(stage_v, acc_sh.at[my_rows])
        plsc.subcore_barrier()

        start_loads(w, idx0, val0, sem0)

        def pair(k2, carry):
            g0 = w + (k2 * 2) * _NW
            g1 = g0 + _NW
            g2 = g1 + _NW

            @pl.when(g1 < n_chunks)
            def _():
                start_loads(g1, idx1, val1, sem1)

            @pl.when(g0 < n_chunks)
            def _():
                wait_loads(idx0, val0, sem0)
                extract_row(idx0, idxf0)
                pltpu.sync_copy(val0, acc_sh.at[idxf0], add=True)

            @pl.when(g2 < n_chunks)
            def _():
                start_loads(g2, idx0, val0, sem0)

            @pl.when(g1 < n_chunks)
            def _():
                wait_loads(idx1, val1, sem1)
                extract_row(idx1, idxf1)
                pltpu.sync_copy(val1, acc_sh.at[idxf1], add=True)

            return carry

        lax.fori_loop(0, (iters + 1) // 2, pair, 0)
        plsc.subcore_barrier()

        pltpu.sync_copy(acc_sh.at[my_rows], stage_v)

        @pl.when(cid == 0)
        def _():
            pltpu.sync_copy(stage_v, out0_hbm.at[my_rows])

        @pl.when(cid == 1)
        def _():
            pltpu.sync_copy(stage_v, out1_hbm.at[my_rows])

    return seg_sum(edge_index, ef_flat)


_P_WR2 = 6 * 128
_P_WL = _P_WR2 + 128
_P_B = _P_WL + 6


def _tc_mlp_body(x_ref, he0_ref, he1_ref, p_ref, wr3_ref, b3c_ref, o_ref):
    d = x_ref.shape[1]
    bn = x_ref.shape[0]
    he = (he0_ref[...] + he1_ref[...]).reshape(bn, 1)
    x = x_ref[...]
    for l in range(6):
        y = jnp.dot(x, p_ref[l * d:(l + 1) * d, :],
                    preferred_element_type=jnp.float32)
        y = y + he * p_ref[_P_WL + l:_P_WL + l + 1, :]
        x = jnp.maximum(y + p_ref[_P_B + l:_P_B + l + 1, :], 0.0)
    x = jnp.maximum(
        jnp.dot(x, p_ref[_P_WR2:_P_WR2 + d, :],
                preferred_element_type=jnp.float32)
        + p_ref[_P_B + 6:_P_B + 7, :], 0.0)
    o_ref[...] = (
        lax.dot_general(wr3_ref[...], x, _DIMS,
                        preferred_element_type=jnp.float32)
        + b3c_ref[...])


def kernel(node_feat, edge_feat, edge_index, W0, b0, W1, b1, W2, b2, W3, b3,
           W4, b4, Wr1, br1, Wr2, br2, Wr3, br3):
    N, D = node_feat.shape
    REG = Wr3.shape[0]

    he0, he1 = _sc_segment_sum(edge_index, edge_feat.reshape(-1))

    sage_w = (W0, W1, W2, W3, W4, Wr1)
    pack = ([jnp.transpose(W[:, :D]) for W in sage_w]
            + [jnp.transpose(Wr2)]
            + [W[:, D].reshape(1, D) for W in sage_w]
            + [b.reshape(1, D) for b in (b0, b1, b2, b3, b4, br1, br2)]
            + [jnp.pad(br3, (0, D - REG)).reshape(1, D)])
    P = jnp.concatenate(pack, axis=0)

    BN = 4096
    grid = (pl.cdiv(N, BN),)
    in_specs = [
        pl.BlockSpec((BN, D), lambda i: (i, 0)),
        pl.BlockSpec((BN,), lambda i: (i,)),
        pl.BlockSpec((BN,), lambda i: (i,)),
        pl.BlockSpec(P.shape, lambda i: (0, 0)),
        pl.BlockSpec((REG, D), lambda i: (0, 0)),
        pl.BlockSpec((REG, 1), lambda i: (0, 0)),
    ]
    out_t = pl.pallas_call(
        _tc_mlp_body,
        grid=grid,
        in_specs=in_specs,
        out_specs=pl.BlockSpec((REG, BN), lambda i: (0, i)),
        out_shape=jax.ShapeDtypeStruct((REG, N), jnp.float32),
        compiler_params=pltpu.CompilerParams(
            dimension_semantics=("arbitrary",)),
    )(node_feat, he0, he1, P, Wr3, br3.reshape(REG, 1))
    return jnp.transpose(out_t)


# speedup vs baseline: 38.6868x; 1.0013x over previous
import functools

import jax
import jax.numpy as jnp
from jax import lax
from jax.experimental import pallas as pl
from jax.experimental.pallas import tpu as pltpu
from jax.experimental.pallas import tpu_sc as plsc

_CHUNK = 6400
_NW = 32
_SLICE = 6272
_NPAD = 16 * _SLICE

_DIMS = (((1,), (1,)), ((), ()))
_OUTER = (((0,), (0,)), ((), ()))


def _sc_segment_sum(edge_index, ef_flat):
    n_chunks = ef_flat.shape[0] // _CHUNK
    iters = (n_chunks + _NW - 1) // _NW
    mesh = plsc.VectorSubcoreMesh(core_axis_name="c", subcore_axis_name="s")

    @functools.partial(
        pl.kernel,
        mesh=mesh,
        out_type=(
            jax.ShapeDtypeStruct((_NPAD,), jnp.float32),
            jax.ShapeDtypeStruct((_NPAD,), jnp.float32),
        ),
        scratch_types=[
            pltpu.VMEM((2, _CHUNK), jnp.int32),
            pltpu.VMEM((_CHUNK,), jnp.float32),
            pltpu.VMEM((2, _CHUNK), jnp.int32),
            pltpu.VMEM((_CHUNK,), jnp.float32),
            pltpu.VMEM((_CHUNK,), jnp.int32),
            pltpu.VMEM((_CHUNK,), jnp.int32),
            pltpu.VMEM((_SLICE,), jnp.float32),
            pltpu.VMEM_SHARED((_NPAD,), jnp.float32),
            pltpu.SemaphoreType.DMA,
            pltpu.SemaphoreType.DMA,
        ],
    )
    def seg_sum(dst_hbm, val_hbm, out0_hbm, out1_hbm,
                idx0, val0, idx1, val1, idxf0, idxf1, stage_v, acc_sh,
                sem0, sem1):
        cid = lax.axis_index("c")
        sid = lax.axis_index("s")
        w = sid * 2 + cid
        my_rows = pl.ds(sid * _SLICE, _SLICE)

        def start_loads(chunk, idx_v, val_v, sem):
            pltpu.async_copy(
                dst_hbm.at[:, pl.ds(chunk * _CHUNK, _CHUNK)], idx_v, sem)
            pltpu.async_copy(
                val_hbm.at[pl.ds(chunk * _CHUNK, _CHUNK)], val_v, sem)

        def wait_loads(idx_v, val_v, sem):
            pltpu.make_async_copy(
                dst_hbm.at[:, pl.ds(0, _CHUNK)], idx_v, sem).wait()
            pltpu.make_async_copy(
                val_hbm.at[pl.ds(0, _CHUNK)], val_v, sem).wait()

        def extract_row(idx_v, idxf_v):
            def cp(i, c):
                for j in range(8):
                    o = (i * 8 + j) * 16
                    idxf_v[pl.ds(o, 16)] = idx_v[1, pl.ds(o, 16)]
                return c

            lax.fori_loop(0, _CHUNK // 128, cp, 0)

        def zero_body(i, carry):
            stage_v[pl.ds(i * 16, 16)] = jnp.zeros((16,), jnp.float32)
            return carry

        lax.fori_loop(0, _SLICE // 16, zero_body, 0)
        pltpu.sync_copy(stage_v, acc_sh.at[my_rows])
        plsc.subcore_barrier()

        start_loads(w, idx0, val0, sem0)

        def pair(k2, carry):
            g0 = w + (k2 * 2) * _NW
            g1 = g0 + _NW
            g2 = g1 + _NW

            @pl.when(g1 < n_chunks)
            def _():
                start_loads(g1, idx1, val1, sem1)

            @pl.when(g0 < n_chunks)
            def _():
                wait_loads(idx0, val0, sem0)
                extract_row(idx0, idxf0)
                pltpu.sync_copy(val0, acc_sh.at[idxf0], add=True)

            @pl.when(g2 < n_chunks)
            def _():
                start_loads(g2, idx0, val0, sem0)

            @pl.when(g1 < n_chunks)
            def _():
                wait_loads(idx1, val1, sem1)
                extract_row(idx1, idxf1)
                pltpu.sync_copy(val1, acc_sh.at[idxf1], add=True)

            return carry

        lax.fori_loop(0, (iters + 1) // 2, pair, 0)
        plsc.subcore_barrier()

        pltpu.sync_copy(acc_sh.at[my_rows], stage_v)

        @pl.when(cid == 0)
        def _():
            pltpu.sync_copy(stage_v, out0_hbm.at[my_rows])

        @pl.when(cid == 1)
        def _():
            pltpu.sync_copy(stage_v, out1_hbm.at[my_rows])

    return seg_sum(edge_index, ef_flat)


_P_WR2 = 6 * 128
_P_WL = _P_WR2 + 128
_P_B = _P_WL + 6


def _tc_mlp_body(x_ref, he0_ref, he1_ref, p_ref, wr3_ref, b3c_ref, o_ref):
    d = x_ref.shape[1]
    bn = x_ref.shape[0]
    he = (he0_ref[...] + he1_ref[...]).reshape(bn, 1)
    x = x_ref[...]
    for l in range(6):
        y = jnp.dot(x, p_ref[l * d:(l + 1) * d, :],
                    preferred_element_type=jnp.float32)
        y = y + he * p_ref[_P_WL + l:_P_WL + l + 1, :]
        x = jnp.maximum(y + p_ref[_P_B + l:_P_B + l + 1, :], 0.0)
    x = jnp.maximum(
        jnp.dot(x, p_ref[_P_WR2:_P_WR2 + d, :],
                preferred_element_type=jnp.float32)
        + p_ref[_P_B + 6:_P_B + 7, :], 0.0)
    o_ref[...] = (
        lax.dot_general(wr3_ref[...], x, _DIMS,
                        preferred_element_type=jnp.float32)
        + b3c_ref[...])


def kernel(node_feat, edge_feat, edge_index, W0, b0, W1, b1, W2, b2, W3, b3,
           W4, b4, Wr1, br1, Wr2, br2, Wr3, br3):
    N, D = node_feat.shape
    REG = Wr3.shape[0]

    he0, he1 = _sc_segment_sum(edge_index, edge_feat.reshape(-1))

    sage_w = (W0, W1, W2, W3, W4, Wr1)
    pack = ([jnp.transpose(W[:, :D]) for W in sage_w]
            + [jnp.transpose(Wr2)]
            + [W[:, D].reshape(1, D) for W in sage_w]
            + [b.reshape(1, D) for b in (b0, b1, b2, b3, b4, br1, br2)]
            + [jnp.pad(br3, (0, D - REG)).reshape(1, D)])
    P = jnp.concatenate(pack, axis=0)

    BN = 8192
    grid = (pl.cdiv(N, BN),)
    in_specs = [
        pl.BlockSpec((BN, D), lambda i: (i, 0)),
        pl.BlockSpec((BN,), lambda i: (i,)),
        pl.BlockSpec((BN,), lambda i: (i,)),
        pl.BlockSpec(P.shape, lambda i: (0, 0)),
        pl.BlockSpec((REG, D), lambda i: (0, 0)),
        pl.BlockSpec((REG, 1), lambda i: (0, 0)),
    ]
    out_t = pl.pallas_call(
        _tc_mlp_body,
        grid=grid,
        in_specs=in_specs,
        out_specs=pl.BlockSpec((REG, BN), lambda i: (0, i)),
        out_shape=jax.ShapeDtypeStruct((REG, N), jnp.float32),
        compiler_params=pltpu.CompilerParams(
            dimension_semantics=("arbitrary",)),
    )(node_feat, he0, he1, P, Wr3, br3.reshape(REG, 1))
    return jnp.transpose(out_t)
